# baseline (speedup 1.0000x reference)
import functools

import jax
import jax.numpy as jnp
from jax import lax
from jax.experimental import pallas as pl
from jax.experimental.pallas import tpu as pltpu
from jax.experimental.pallas import tpu_sc as plsc

BS, L, D, H = 64, 200, 128, 128
NTOK = BS * L
L2_CONST = 1e-05
NC, NS = 2, 16
NW = NC * NS
BPW = NTOK // NW

_f32 = jnp.float32
_i32 = jnp.int32


@functools.lru_cache(maxsize=1)
def _sc_gather_fn():
    mesh = plsc.VectorSubcoreMesh(core_axis_name="c", subcore_axis_name="s")

    @functools.partial(
        pl.kernel,
        mesh=mesh,
        out_type=[
            jax.ShapeDtypeStruct((NTOK,), _f32),
            jax.ShapeDtypeStruct((NTOK, D), _f32),
            jax.ShapeDtypeStruct((NTOK, D), _f32),
            jax.ShapeDtypeStruct((NTOK, D), _f32),
            jax.ShapeDtypeStruct((NTOK,), _f32),
        ],
        scratch_types=[
            pltpu.VMEM((BPW,), _i32),
            pltpu.VMEM((BPW,), _i32),
            pltpu.VMEM((BPW,), _i32),
            pltpu.VMEM((BPW,), _f32),
            pltpu.VMEM((BPW,), _f32),
            pltpu.VMEM((BPW, D), _f32),
            pltpu.VMEM((BPW, D), _f32),
            pltpu.SemaphoreType.DMA,
            pltpu.SemaphoreType.DMA,
            pltpu.SemaphoreType.DMA,
            pltpu.SemaphoreType.DMA,
            pltpu.SemaphoreType.DMA,
            pltpu.SemaphoreType.DMA,
            pltpu.SemaphoreType.DMA,
            pltpu.SemaphoreType.DMA,
        ],
    )
    def _gather(pid_hbm, q_hbm, qm1_hbm, diff_hbm, qemb_hbm, qed_hbm, m2_hbm,
                c2_hbm, pid_out, qe_out, qed_out, m2g_out, c2g_out,
                idx_p, idx_q, idx_m, vals1, vals2, rows1, rows2,
                s_ip, s_iq, s_im, s_a, s_b, s_c, s_d, s_e):
        wid = lax.axis_index("s") * NC + lax.axis_index("c")
        base = wid * BPW
        sl = pl.ds(base, BPW)
        d_ip = pltpu.async_copy(pid_hbm.at[sl], idx_p, s_ip)
        d_iq = pltpu.async_copy(q_hbm.at[sl], idx_q, s_iq)
        d_im = pltpu.async_copy(qm1_hbm.at[sl], idx_m, s_im)
        d_ip.wait()
        g_a = pltpu.async_copy(diff_hbm.at[idx_p], vals1, s_a)
        d_iq.wait()
        g_b = pltpu.async_copy(qemb_hbm.at[idx_q], rows1, s_b)
        g_c = pltpu.async_copy(qed_hbm.at[idx_q], rows2, s_c)
        d_im.wait()
        g_e = pltpu.async_copy(c2_hbm.at[idx_m], vals2, s_e)
        g_a.wait()
        pltpu.sync_copy(vals1, pid_out.at[sl])
        g_b.wait()
        pltpu.sync_copy(rows1, qe_out.at[sl])
        g_d = pltpu.async_copy(m2_hbm.at[idx_m], rows1, s_d)
        g_c.wait()
        pltpu.sync_copy(rows2, qed_out.at[sl])
        g_e.wait()
        pltpu.sync_copy(vals2, c2g_out.at[sl])
        g_d.wait()
        pltpu.sync_copy(rows1, m2g_out.at[sl])

    return _gather


def _sc_gather(pidT, qT, qm1, diff_flat, q_emb, q_emb_diff, M2, c2_flat):
    return _sc_gather_fn()(pidT, qT, qm1, diff_flat, q_emb, q_emb_diff,
                           M2, c2_flat)


def _m2_body(mat_ref, fcw_ref, fcb_ref, m2_ref, c2_ref):
    mat = mat_ref[...]
    m2_ref[...] = jnp.dot(mat, fcw_ref[...], preferred_element_type=_f32)
    c2_ref[...] = jnp.dot(mat, fcb_ref[...], preferred_element_type=_f32)


def _m2_call(matrix, fc_W, fc_b2):
    nq = matrix.shape[0]
    return pl.pallas_call(
        _m2_body,
        out_shape=[jax.ShapeDtypeStruct((nq, H), _f32),
                   jax.ShapeDtypeStruct((nq, 1), _f32)],
    )(matrix, fc_W, fc_b2)


_T_BLK = 4
_RB = _T_BLK * BS


def _gru_body(qe_ref, qed_ref, pid_ref, qa_ref, m2g_ref, c2g_ref, qaemb_ref,
              qadiff_ref, wih_ref, bcomb_ref, whh_ref, bhn_ref,
              m_ref, pidsq_ref, h_ref):
    @pl.when(pl.program_id(0) == 0)
    def _():
        h_ref[...] = jnp.zeros_like(h_ref)
        pidsq_ref[...] = jnp.zeros_like(pidsq_ref)

    qe = qe_ref[...]
    qed = qed_ref[...]
    pid = pid_ref[...]
    qa1 = qa_ref[...] == 1
    row0 = qaemb_ref[0:1, :]
    row1 = qaemb_ref[1:2, :]
    d0 = qadiff_ref[0:1, :]
    d1 = qadiff_ref[1:2, :]
    qa_row = jnp.where(qa1, row1, row0)
    qa_diff = jnp.where(qa1, d1, d0)
    q_full = qe + pid * qed
    qa_full = qe + qa_row + pid * qa_diff
    x = jnp.concatenate([qa_full, q_full], axis=1)
    gi_all = jnp.dot(x, wih_ref[...],
                     preferred_element_type=_f32) + bcomb_ref[...]
    pidsq_ref[...] += jnp.sum(pid * pid).reshape(1, 1)

    whh = whh_ref[...]
    bhn = bhn_ref[...]
    hb = BS // 2
    h1 = h_ref[0:hb, :]
    h2 = h_ref[hb:BS, :]

    def _step(h, gh, gi):
        r = jax.nn.sigmoid(gi[:, :H] + gh[:, :H])
        z = jax.nn.sigmoid(gi[:, H:2 * H] + gh[:, H:2 * H])
        n = jnp.tanh(gi[:, 2 * H:] + r * (gh[:, 2 * H:] + bhn))
        return (1.0 - z) * n + z * h

    for j in range(_T_BLK):
        base = j * BS
        gh1 = jnp.dot(h1, whh, preferred_element_type=_f32)
        gh2 = jnp.dot(h2, whh, preferred_element_type=_f32)
        h1 = _step(h1, gh1, gi_all[base:base + hb])
        h2 = _step(h2, gh2, gi_all[base + hb:base + BS])
        m_ref[base:base + hb] = (
            jnp.sum(h1 * m2g_ref[base:base + hb], axis=1, keepdims=True)
            + c2g_ref[base:base + hb])
        m_ref[base + hb:base + BS] = (
            jnp.sum(h2 * m2g_ref[base + hb:base + BS], axis=1, keepdims=True)
            + c2g_ref[base + hb:base + BS])
    h_ref[0:hb, :] = h1
    h_ref[hb:BS, :] = h2


def _gru_call(qe, qed, pid2, qa2, M2g, c2g2, qa_emb, qa_diff2, W_ihT, b_comb,
              W_hhT, bhn2):
    blk = lambda w: pl.BlockSpec((_RB, w), lambda t: (t, 0))
    full = lambda a, b: pl.BlockSpec((a, b), lambda t: (0, 0))
    return pl.pallas_call(
        _gru_body,
        grid=(L // _T_BLK,),
        in_specs=[
            blk(D), blk(D), blk(1), blk(1), blk(D), blk(1),
            full(2, D), full(2, D), full(2 * D, 3 * H), full(1, 3 * H),
            full(H, 3 * H), full(1, H),
        ],
        out_specs=[
            blk(1),
            pl.BlockSpec((1, 1), lambda t: (0, 0)),
        ],
        out_shape=[jax.ShapeDtypeStruct((NTOK, 1), _f32),
                   jax.ShapeDtypeStruct((1, 1), _f32)],
        scratch_shapes=[pltpu.VMEM((BS, H), _f32)],
    )(qe, qed, pid2, qa2, M2g, c2g2, qa_emb, qa_diff2, W_ihT, b_comb,
      W_hhT, bhn2)


_S_BLK = 8


def _student_body(m_col_ref, m_row_ref, q_col_ref, q_row_ref, qa_col_ref,
                  qa_row_ref, tgt_ref, pidsq_ref, pred_ref, loss_ref, cnt_ref):
    s = pl.program_id(0)
    m_col = m_col_ref[...]
    m_row = m_row_ref[...]
    q_col = q_col_ref[...]
    q_row = q_row_ref[...]
    qa_col = qa_col_ref[...]
    qa_row = qa_row_ref[...]

    mk_col = jnp.where(m_col >= 0.4, 1.0, m_col)
    mk_row = jnp.where(m_row >= 0.4, 1.0, m_row)
    i1c = mk_col == 1.0
    i0c = mk_col == 0.0
    i1r = mk_row == 1.0
    qa1c = qa_col == 1
    qa1r = qa_row == 1

    eq = q_col == q_row
    ri = lax.broadcasted_iota(_i32, (_S_BLK, L, L), 2)
    ci = lax.broadcasted_iota(_i32, (_S_BLK, L, L), 1)
    m_lt = eq & (ri < ci)
    eq_le = eq & (ri <= ci)

    w_mc = jnp.where(qa1c & i1c, 1.0, 0.0)
    w_mi = jnp.where((~qa1c) & i1c, 1.0 / 256.0, 0.0)
    w_nmc = jnp.where((~qa1c) & i0c, 1.0 / 65536.0, 0.0)
    w_packed = (w_mc + w_mi + w_nmc).reshape(_S_BLK, 1, L)
    v = jnp.sum(jnp.where(m_lt, w_packed, 0.0), axis=2, keepdims=True)
    aa = jnp.sum(jnp.where(m_lt, 1.0, 0.0), axis=2, keepdims=True) + 1.0
    mc = jnp.floor(v)
    r1 = (v - mc) * 256.0
    mi = jnp.floor(r1)
    nmc = (r1 - mi) * 256.0

    g_val = jnp.where(i1c & qa1c, mc / aa,
                      jnp.where((~i1c) & (~qa1c), 1.0 - nmc / aa, nmc / aa))
    s_val = mi / aa

    kf = lax.broadcasted_iota(
        _i32, (_S_BLK, 1, L), 2).astype(_f32) * 2.0
    g_enc = kf + g_val.reshape(_S_BLK, 1, L)
    s_enc = kf + s_val.reshape(_S_BLK, 1, L)
    set_g_r = (i1r & qa1r) | (~i1r)
    set_s_r = i1r & (~qa1r)
    ge = jnp.max(jnp.where(eq_le & set_g_r, g_enc, -1.0), axis=2,
                 keepdims=True)
    se = jnp.max(jnp.where(eq_le & set_s_r, s_enc, -1.0), axis=2,
                 keepdims=True)
    guess = jnp.where(ge < 0.0, 0.0, ge - 2.0 * jnp.floor(ge * 0.5))
    slip = jnp.where(se < 0.0, 0.0, se - 2.0 * jnp.floor(se * 0.5))

    res = (1.0 - slip) * (mk_col * guess + (1.0 - slip) * (1.0 - mk_col))
    pred_ref[...] = jax.nn.sigmoid(res)

    tgt = tgt_ref[...]
    maskl = tgt > -0.9
    sq = (res - tgt) * (res - tgt)

    @pl.when(s == 0)
    def _():
        loss_ref[...] = L2_CONST * pidsq_ref[...]
        cnt_ref[...] = jnp.zeros_like(cnt_ref)

    loss_ref[...] += jnp.sum(jnp.where(maskl, sq, 0.0)).reshape(1, 1)
    cnt_ref[...] += jnp.sum(maskl.astype(_i32)).reshape(1, 1)


def _student_call(m_col3, m_row3, q_col3, q_row3, qa_col3, qa_row3, tgt_col3,
                  pidsq):
    col = pl.BlockSpec((_S_BLK, L, 1), lambda s: (s, 0, 0))
    row = pl.BlockSpec((_S_BLK, 1, L), lambda s: (s, 0, 0))
    return pl.pallas_call(
        _student_body,
        grid=(BS // _S_BLK,),
        in_specs=[col, row, col, row, col, row, col,
                  pl.BlockSpec((1, 1), lambda s: (0, 0))],
        out_specs=[col,
                   pl.BlockSpec((1, 1), lambda s: (0, 0)),
                   pl.BlockSpec((1, 1), lambda s: (0, 0))],
        out_shape=[jax.ShapeDtypeStruct((BS, L, 1), _f32),
                   jax.ShapeDtypeStruct((1, 1), _f32),
                   jax.ShapeDtypeStruct((1, 1), _i32)],
    )(m_col3, m_row3, q_col3, q_row3, qa_col3, qa_row3, tgt_col3, pidsq)


def kernel(q_data, qa_data, matrix, target, pid_data, q_emb, qa_emb, diff_parm,
           q_emb_diff, qa_emb_diff, W_ih, W_hh, b_ih, b_hh, fc_W, fc_b):
    nq = q_emb.shape[0] - 1
    q_i = q_data.astype(_i32)
    qa = (qa_data.astype(_i32) - q_i) // nq
    qT = q_i.T.reshape(-1)
    pidT = pid_data.astype(_i32).T.reshape(-1)
    qm1 = qT - 1
    diff_flat = diff_parm.reshape(-1)

    M2, c2 = _m2_call(matrix, fc_W, fc_b.reshape(-1, 1))
    pid_e, qe, qed, M2g, c2g = _sc_gather(
        pidT, qT, qm1, diff_flat, q_emb, q_emb_diff, M2, c2.reshape(-1))

    b_comb = (b_ih + jnp.concatenate(
        [b_hh[:2 * H], jnp.zeros((H,), _f32)])).reshape(1, -1)
    m3, pidsq = _gru_call(qe, qed, pid_e.reshape(-1, 1), qa.T.reshape(-1, 1),
                          M2g, c2g.reshape(-1, 1), qa_emb, qa_emb_diff[:2],
                          W_ih.T, b_comb, W_hh.T, b_hh[2 * H:].reshape(1, H))

    m_bl = m3.reshape(L, BS).T
    preds3, loss, cnt = _student_call(
        m_bl.reshape(BS, L, 1), m_bl.reshape(BS, 1, L),
        q_i.reshape(BS, L, 1), q_i.reshape(BS, 1, L),
        qa.reshape(BS, L, 1), qa.reshape(BS, 1, L),
        target.reshape(BS, L, 1), pidsq)

    preds = preds3.reshape(-1)
    return loss[0, 0], preds, cnt[0, 0]


# speedup vs baseline: 1.0845x; 1.0845x over previous
import functools

import jax
import jax.numpy as jnp
from jax import lax
from jax.experimental import pallas as pl
from jax.experimental.pallas import tpu as pltpu
from jax.experimental.pallas import tpu_sc as plsc

BS, L, D, H = 64, 200, 128, 128
NTOK = BS * L
L2_CONST = 1e-05
NC, NS = 2, 16
NW = NC * NS
BPW = NTOK // NW

_f32 = jnp.float32
_i32 = jnp.int32


@functools.lru_cache(maxsize=1)
def _sc_gather_fn():
    mesh = plsc.VectorSubcoreMesh(core_axis_name="c", subcore_axis_name="s")

    @functools.partial(
        pl.kernel,
        mesh=mesh,
        out_type=[
            jax.ShapeDtypeStruct((NTOK,), _f32),
            jax.ShapeDtypeStruct((NTOK, D), _f32),
            jax.ShapeDtypeStruct((NTOK, D), _f32),
            jax.ShapeDtypeStruct((NTOK, D), _f32),
            jax.ShapeDtypeStruct((NTOK,), _f32),
        ],
        scratch_types=[
            pltpu.VMEM((BPW,), _i32),
            pltpu.VMEM((BPW,), _i32),
            pltpu.VMEM((BPW,), _i32),
            pltpu.VMEM((BPW,), _f32),
            pltpu.VMEM((BPW,), _f32),
            pltpu.VMEM((BPW, D), _f32),
            pltpu.VMEM((BPW, D), _f32),
            pltpu.SemaphoreType.DMA,
            pltpu.SemaphoreType.DMA,
            pltpu.SemaphoreType.DMA,
            pltpu.SemaphoreType.DMA,
            pltpu.SemaphoreType.DMA,
            pltpu.SemaphoreType.DMA,
            pltpu.SemaphoreType.DMA,
            pltpu.SemaphoreType.DMA,
        ],
    )
    def _gather(pid_hbm, q_hbm, qm1_hbm, diff_hbm, qemb_hbm, qed_hbm, m2_hbm,
                c2_hbm, pid_out, qe_out, qed_out, m2g_out, c2g_out,
                idx_p, idx_q, idx_m, vals1, vals2, rows1, rows2,
                s_ip, s_iq, s_im, s_a, s_b, s_c, s_d, s_e):
        wid = lax.axis_index("s") * NC + lax.axis_index("c")
        base = wid * BPW
        sl = pl.ds(base, BPW)
        d_ip = pltpu.async_copy(pid_hbm.at[sl], idx_p, s_ip)
        d_iq = pltpu.async_copy(q_hbm.at[sl], idx_q, s_iq)
        d_im = pltpu.async_copy(qm1_hbm.at[sl], idx_m, s_im)
        d_ip.wait()
        g_a = pltpu.async_copy(diff_hbm.at[idx_p], vals1, s_a)
        d_iq.wait()
        g_b = pltpu.async_copy(qemb_hbm.at[idx_q], rows1, s_b)
        g_c = pltpu.async_copy(qed_hbm.at[idx_q], rows2, s_c)
        d_im.wait()
        g_e = pltpu.async_copy(c2_hbm.at[idx_m], vals2, s_e)
        g_a.wait()
        pltpu.sync_copy(vals1, pid_out.at[sl])
        g_b.wait()
        pltpu.sync_copy(rows1, qe_out.at[sl])
        g_d = pltpu.async_copy(m2_hbm.at[idx_m], rows1, s_d)
        g_c.wait()
        pltpu.sync_copy(rows2, qed_out.at[sl])
        g_e.wait()
        pltpu.sync_copy(vals2, c2g_out.at[sl])
        g_d.wait()
        pltpu.sync_copy(rows1, m2g_out.at[sl])

    return _gather


def _sc_gather(pidT, qT, qm1, diff_flat, q_emb, q_emb_diff, M2, c2_flat):
    return _sc_gather_fn()(pidT, qT, qm1, diff_flat, q_emb, q_emb_diff,
                           M2, c2_flat)


def _m2_body(mat_ref, fcw_ref, fcb_ref, m2_ref, c2_ref):
    mat = mat_ref[...]
    m2_ref[...] = jnp.dot(mat, fcw_ref[...], preferred_element_type=_f32)
    c2_ref[...] = jnp.dot(mat, fcb_ref[...], preferred_element_type=_f32)


def _m2_call(matrix, fc_W, fc_b2):
    nq = matrix.shape[0]
    return pl.pallas_call(
        _m2_body,
        out_shape=[jax.ShapeDtypeStruct((nq, H), _f32),
                   jax.ShapeDtypeStruct((nq, 1), _f32)],
    )(matrix, fc_W, fc_b2)


_T_BLK = 8
_RB = _T_BLK * BS


def _gru_body(qe_ref, qed_ref, pid_ref, qa_ref, m2g_ref, c2g_ref, qaemb_ref,
              qadiff_ref, wih_ref, bcomb_ref, whh_ref, bhn_ref,
              m_ref, pidsq_ref, h_ref):
    @pl.when(pl.program_id(0) == 0)
    def _():
        h_ref[...] = jnp.zeros_like(h_ref)
        pidsq_ref[...] = jnp.zeros_like(pidsq_ref)

    qe = qe_ref[...]
    qed = qed_ref[...]
    pid = pid_ref[...]
    qa1 = qa_ref[...] == 1
    row0 = qaemb_ref[0:1, :]
    row1 = qaemb_ref[1:2, :]
    d0 = qadiff_ref[0:1, :]
    d1 = qadiff_ref[1:2, :]
    qa_row = jnp.where(qa1, row1, row0)
    qa_diff = jnp.where(qa1, d1, d0)
    q_full = qe + pid * qed
    qa_full = qe + qa_row + pid * qa_diff
    x = jnp.concatenate([qa_full, q_full], axis=1)
    gi_all = jnp.dot(x, wih_ref[...],
                     preferred_element_type=_f32) + bcomb_ref[...]
    pidsq_ref[...] += jnp.sum(pid * pid).reshape(1, 1)

    whh = whh_ref[...]
    bhn = bhn_ref[...]
    hb = BS // 2
    h1 = h_ref[0:hb, :]
    h2 = h_ref[hb:BS, :]

    def _step(h, gh, gi):
        r = jax.nn.sigmoid(gi[:, :H] + gh[:, :H])
        z = jax.nn.sigmoid(gi[:, H:2 * H] + gh[:, H:2 * H])
        n = jnp.tanh(gi[:, 2 * H:] + r * (gh[:, 2 * H:] + bhn))
        return (1.0 - z) * n + z * h

    for j in range(_T_BLK):
        base = j * BS
        gh1 = jnp.dot(h1, whh, preferred_element_type=_f32)
        gh2 = jnp.dot(h2, whh, preferred_element_type=_f32)
        h1 = _step(h1, gh1, gi_all[base:base + hb])
        h2 = _step(h2, gh2, gi_all[base + hb:base + BS])
        m_ref[base:base + hb] = (
            jnp.sum(h1 * m2g_ref[base:base + hb], axis=1, keepdims=True)
            + c2g_ref[base:base + hb])
        m_ref[base + hb:base + BS] = (
            jnp.sum(h2 * m2g_ref[base + hb:base + BS], axis=1, keepdims=True)
            + c2g_ref[base + hb:base + BS])
    h_ref[0:hb, :] = h1
    h_ref[hb:BS, :] = h2


def _gru_call(qe, qed, pid2, qa2, M2g, c2g2, qa_emb, qa_diff2, W_ihT, b_comb,
              W_hhT, bhn2):
    blk = lambda w: pl.BlockSpec((_RB, w), lambda t: (t, 0))
    full = lambda a, b: pl.BlockSpec((a, b), lambda t: (0, 0))
    return pl.pallas_call(
        _gru_body,
        grid=(L // _T_BLK,),
        in_specs=[
            blk(D), blk(D), blk(1), blk(1), blk(D), blk(1),
            full(2, D), full(2, D), full(2 * D, 3 * H), full(1, 3 * H),
            full(H, 3 * H), full(1, H),
        ],
        out_specs=[
            blk(1),
            pl.BlockSpec((1, 1), lambda t: (0, 0)),
        ],
        out_shape=[jax.ShapeDtypeStruct((NTOK, 1), _f32),
                   jax.ShapeDtypeStruct((1, 1), _f32)],
        scratch_shapes=[pltpu.VMEM((BS, H), _f32)],
    )(qe, qed, pid2, qa2, M2g, c2g2, qa_emb, qa_diff2, W_ihT, b_comb,
      W_hhT, bhn2)


_S_BLK = 8


def _student_body(m_col_ref, m_row_ref, q_col_ref, q_row_ref, qa_col_ref,
                  qa_row_ref, tgt_ref, pidsq_ref, pred_ref, loss_ref, cnt_ref):
    s = pl.program_id(0)
    m_col = m_col_ref[...]
    m_row = m_row_ref[...]
    q_col = q_col_ref[...]
    q_row = q_row_ref[...]
    qa_col = qa_col_ref[...]
    qa_row = qa_row_ref[...]

    mk_col = jnp.where(m_col >= 0.4, 1.0, m_col)
    mk_row = jnp.where(m_row >= 0.4, 1.0, m_row)
    i1c = mk_col == 1.0
    i0c = mk_col == 0.0
    i1r = mk_row == 1.0
    qa1c = qa_col == 1
    qa1r = qa_row == 1

    eq = q_col == q_row
    ri = lax.broadcasted_iota(_i32, (_S_BLK, L, L), 2)
    ci = lax.broadcasted_iota(_i32, (_S_BLK, L, L), 1)
    m_lt = eq & (ri < ci)
    eq_le = eq & (ri <= ci)

    w_mc = jnp.where(qa1c & i1c, 1.0, 0.0)
    w_mi = jnp.where((~qa1c) & i1c, 1.0 / 256.0, 0.0)
    w_nmc = jnp.where((~qa1c) & i0c, 1.0 / 65536.0, 0.0)
    w_packed = (w_mc + w_mi + w_nmc).reshape(_S_BLK, 1, L)
    v = jnp.sum(jnp.where(m_lt, w_packed, 0.0), axis=2, keepdims=True)
    aa = jnp.sum(jnp.where(m_lt, 1.0, 0.0), axis=2, keepdims=True) + 1.0
    mc = jnp.floor(v)
    r1 = (v - mc) * 256.0
    mi = jnp.floor(r1)
    nmc = (r1 - mi) * 256.0

    g_val = jnp.where(i1c & qa1c, mc / aa,
                      jnp.where((~i1c) & (~qa1c), 1.0 - nmc / aa, nmc / aa))
    s_val = mi / aa

    kf = lax.broadcasted_iota(
        _i32, (_S_BLK, 1, L), 2).astype(_f32) * 2.0
    g_enc = kf + g_val.reshape(_S_BLK, 1, L)
    s_enc = kf + s_val.reshape(_S_BLK, 1, L)
    set_g_r = (i1r & qa1r) | (~i1r)
    set_s_r = i1r & (~qa1r)
    ge = jnp.max(jnp.where(eq_le & set_g_r, g_enc, -1.0), axis=2,
                 keepdims=True)
    se = jnp.max(jnp.where(eq_le & set_s_r, s_enc, -1.0), axis=2,
                 keepdims=True)
    guess = jnp.where(ge < 0.0, 0.0, ge - 2.0 * jnp.floor(ge * 0.5))
    slip = jnp.where(se < 0.0, 0.0, se - 2.0 * jnp.floor(se * 0.5))

    res = (1.0 - slip) * (mk_col * guess + (1.0 - slip) * (1.0 - mk_col))
    pred_ref[...] = jax.nn.sigmoid(res)

    tgt = tgt_ref[...]
    maskl = tgt > -0.9
    sq = (res - tgt) * (res - tgt)

    @pl.when(s == 0)
    def _():
        loss_ref[...] = L2_CONST * pidsq_ref[...]
        cnt_ref[...] = jnp.zeros_like(cnt_ref)

    loss_ref[...] += jnp.sum(jnp.where(maskl, sq, 0.0)).reshape(1, 1)
    cnt_ref[...] += jnp.sum(maskl.astype(_i32)).reshape(1, 1)


def _student_call(m_col3, m_row3, q_col3, q_row3, qa_col3, qa_row3, tgt_col3,
                  pidsq):
    col = pl.BlockSpec((_S_BLK, L, 1), lambda s: (s, 0, 0))
    row = pl.BlockSpec((_S_BLK, 1, L), lambda s: (s, 0, 0))
    return pl.pallas_call(
        _student_body,
        grid=(BS // _S_BLK,),
        in_specs=[col, row, col, row, col, row, col,
                  pl.BlockSpec((1, 1), lambda s: (0, 0))],
        out_specs=[col,
                   pl.BlockSpec((1, 1), lambda s: (0, 0)),
                   pl.BlockSpec((1, 1), lambda s: (0, 0))],
        out_shape=[jax.ShapeDtypeStruct((BS, L, 1), _f32),
                   jax.ShapeDtypeStruct((1, 1), _f32),
                   jax.ShapeDtypeStruct((1, 1), _i32)],
    )(m_col3, m_row3, q_col3, q_row3, qa_col3, qa_row3, tgt_col3, pidsq)


def kernel(q_data, qa_data, matrix, target, pid_data, q_emb, qa_emb, diff_parm,
           q_emb_diff, qa_emb_diff, W_ih, W_hh, b_ih, b_hh, fc_W, fc_b):
    nq = q_emb.shape[0] - 1
    q_i = q_data.astype(_i32)
    qa = (qa_data.astype(_i32) - q_i) // nq
    qT = q_i.T.reshape(-1)
    pidT = pid_data.astype(_i32).T.reshape(-1)
    qm1 = qT - 1
    diff_flat = diff_parm.reshape(-1)

    M2, c2 = _m2_call(matrix, fc_W, fc_b.reshape(-1, 1))
    pid_e, qe, qed, M2g, c2g = _sc_gather(
        pidT, qT, qm1, diff_flat, q_emb, q_emb_diff, M2, c2.reshape(-1))

    b_comb = (b_ih + jnp.concatenate(
        [b_hh[:2 * H], jnp.zeros((H,), _f32)])).reshape(1, -1)
    m3, pidsq = _gru_call(qe, qed, pid_e.reshape(-1, 1), qa.T.reshape(-1, 1),
                          M2g, c2g.reshape(-1, 1), qa_emb, qa_emb_diff[:2],
                          W_ih.T, b_comb, W_hh.T, b_hh[2 * H:].reshape(1, H))

    m_bl = m3.reshape(L, BS).T
    preds3, loss, cnt = _student_call(
        m_bl.reshape(BS, L, 1), m_bl.reshape(BS, 1, L),
        q_i.reshape(BS, L, 1), q_i.reshape(BS, 1, L),
        qa.reshape(BS, L, 1), qa.reshape(BS, 1, L),
        target.reshape(BS, L, 1), pidsq)

    preds = preds3.reshape(-1)
    return loss[0, 0], preds, cnt[0, 0]


# speedup vs baseline: 1.1096x; 1.0232x over previous
import functools

import jax
import jax.numpy as jnp
from jax import lax
from jax.experimental import pallas as pl
from jax.experimental.pallas import tpu as pltpu
from jax.experimental.pallas import tpu_sc as plsc

BS, L, D, H = 64, 200, 128, 128
NTOK = BS * L
L2_CONST = 1e-05
NC, NS = 2, 16
NW = NC * NS
BPW = NTOK // NW

_f32 = jnp.float32
_i32 = jnp.int32


@functools.lru_cache(maxsize=1)
def _sc_gather_fn():
    mesh = plsc.VectorSubcoreMesh(core_axis_name="c", subcore_axis_name="s")

    @functools.partial(
        pl.kernel,
        mesh=mesh,
        out_type=[
            jax.ShapeDtypeStruct((NTOK,), _f32),
            jax.ShapeDtypeStruct((NTOK, D), _f32),
            jax.ShapeDtypeStruct((NTOK, D), _f32),
            jax.ShapeDtypeStruct((NTOK, D), _f32),
            jax.ShapeDtypeStruct((NTOK,), _f32),
        ],
        scratch_types=[
            pltpu.VMEM((BPW,), _i32),
            pltpu.VMEM((BPW,), _i32),
            pltpu.VMEM((BPW,), _i32),
            pltpu.VMEM((BPW,), _f32),
            pltpu.VMEM((BPW,), _f32),
            pltpu.VMEM((BPW, D), _f32),
            pltpu.VMEM((BPW, D), _f32),
            pltpu.SemaphoreType.DMA,
            pltpu.SemaphoreType.DMA,
            pltpu.SemaphoreType.DMA,
            pltpu.SemaphoreType.DMA,
            pltpu.SemaphoreType.DMA,
            pltpu.SemaphoreType.DMA,
            pltpu.SemaphoreType.DMA,
            pltpu.SemaphoreType.DMA,
        ],
    )
    def _gather(pid_hbm, q_hbm, qm1_hbm, diff_hbm, qemb_hbm, qed_hbm, m2_hbm,
                c2_hbm, pid_out, qe_out, qed_out, m2g_out, c2g_out,
                idx_p, idx_q, idx_m, vals1, vals2, rows1, rows2,
                s_ip, s_iq, s_im, s_a, s_b, s_c, s_d, s_e):
        wid = lax.axis_index("s") * NC + lax.axis_index("c")
        base = wid * BPW
        sl = pl.ds(base, BPW)
        d_ip = pltpu.async_copy(pid_hbm.at[sl], idx_p, s_ip)
        d_iq = pltpu.async_copy(q_hbm.at[sl], idx_q, s_iq)
        d_im = pltpu.async_copy(qm1_hbm.at[sl], idx_m, s_im)
        d_ip.wait()
        g_a = pltpu.async_copy(diff_hbm.at[idx_p], vals1, s_a)
        d_iq.wait()
        g_b = pltpu.async_copy(qemb_hbm.at[idx_q], rows1, s_b)
        g_c = pltpu.async_copy(qed_hbm.at[idx_q], rows2, s_c)
        d_im.wait()
        g_e = pltpu.async_copy(c2_hbm.at[idx_m], vals2, s_e)
        g_a.wait()
        pltpu.sync_copy(vals1, pid_out.at[sl])
        g_b.wait()
        pltpu.sync_copy(rows1, qe_out.at[sl])
        g_d = pltpu.async_copy(m2_hbm.at[idx_m], rows1, s_d)
        g_c.wait()
        pltpu.sync_copy(rows2, qed_out.at[sl])
        g_e.wait()
        pltpu.sync_copy(vals2, c2g_out.at[sl])
        g_d.wait()
        pltpu.sync_copy(rows1, m2g_out.at[sl])

    return _gather


def _sc_gather(pidT, qT, qm1, diff_flat, q_emb, q_emb_diff, M2, c2_flat):
    return _sc_gather_fn()(pidT, qT, qm1, diff_flat, q_emb, q_emb_diff,
                           M2, c2_flat)


def _m2_body(mat_ref, fcw_ref, fcb_ref, m2_ref, c2_ref):
    mat = mat_ref[...]
    m2_ref[...] = jnp.dot(mat, fcw_ref[...], preferred_element_type=_f32)
    c2_ref[...] = jnp.dot(mat, fcb_ref[...], preferred_element_type=_f32)


def _m2_call(matrix, fc_W, fc_b2):
    nq = matrix.shape[0]
    return pl.pallas_call(
        _m2_body,
        out_shape=[jax.ShapeDtypeStruct((nq, H), _f32),
                   jax.ShapeDtypeStruct((nq, 1), _f32)],
    )(matrix, fc_W, fc_b2)


_T_BLK = 20
_RB = _T_BLK * BS


def _gru_body(qe_ref, qed_ref, pid_ref, qa_ref, m2g_ref, c2g_ref, qaemb_ref,
              qadiff_ref, wih_ref, bcomb_ref, whh_ref, bhn_ref,
              m_ref, pidsq_ref, h_ref):
    @pl.when(pl.program_id(0) == 0)
    def _():
        h_ref[...] = jnp.zeros_like(h_ref)
        pidsq_ref[...] = jnp.zeros_like(pidsq_ref)

    qe = qe_ref[...]
    qed = qed_ref[...]
    pid = pid_ref[...]
    qa1 = qa_ref[...] == 1
    row0 = qaemb_ref[0:1, :]
    row1 = qaemb_ref[1:2, :]
    d0 = qadiff_ref[0:1, :]
    d1 = qadiff_ref[1:2, :]
    qa_row = jnp.where(qa1, row1, row0)
    qa_diff = jnp.where(qa1, d1, d0)
    q_full = qe + pid * qed
    qa_full = qe + qa_row + pid * qa_diff
    x = jnp.concatenate([qa_full, q_full], axis=1)
    gi_all = jnp.dot(x, wih_ref[...],
                     preferred_element_type=_f32) + bcomb_ref[...]
    pidsq_ref[...] += jnp.sum(pid * pid).reshape(1, 1)

    whh = whh_ref[...]
    bhn = bhn_ref[...]
    hb = BS // 2
    h1 = h_ref[0:hb, :]
    h2 = h_ref[hb:BS, :]

    def _step(h, gh, gi):
        r = jax.nn.sigmoid(gi[:, :H] + gh[:, :H])
        z = jax.nn.sigmoid(gi[:, H:2 * H] + gh[:, H:2 * H])
        n = jnp.tanh(gi[:, 2 * H:] + r * (gh[:, 2 * H:] + bhn))
        return (1.0 - z) * n + z * h

    for j in range(_T_BLK):
        base = j * BS
        gh1 = jnp.dot(h1, whh, preferred_element_type=_f32)
        gh2 = jnp.dot(h2, whh, preferred_element_type=_f32)
        h1 = _step(h1, gh1, gi_all[base:base + hb])
        h2 = _step(h2, gh2, gi_all[base + hb:base + BS])
        m_ref[base:base + hb] = (
            jnp.sum(h1 * m2g_ref[base:base + hb], axis=1, keepdims=True)
            + c2g_ref[base:base + hb])
        m_ref[base + hb:base + BS] = (
            jnp.sum(h2 * m2g_ref[base + hb:base + BS], axis=1, keepdims=True)
            + c2g_ref[base + hb:base + BS])
    h_ref[0:hb, :] = h1
    h_ref[hb:BS, :] = h2


def _gru_call(qe, qed, pid2, qa2, M2g, c2g2, qa_emb, qa_diff2, W_ihT, b_comb,
              W_hhT, bhn2):
    blk = lambda w: pl.BlockSpec((_RB, w), lambda t: (t, 0))
    full = lambda a, b: pl.BlockSpec((a, b), lambda t: (0, 0))
    return pl.pallas_call(
        _gru_body,
        grid=(L // _T_BLK,),
        in_specs=[
            blk(D), blk(D), blk(1), blk(1), blk(D), blk(1),
            full(2, D), full(2, D), full(2 * D, 3 * H), full(1, 3 * H),
            full(H, 3 * H), full(1, H),
        ],
        out_specs=[
            blk(1),
            pl.BlockSpec((1, 1), lambda t: (0, 0)),
        ],
        out_shape=[jax.ShapeDtypeStruct((NTOK, 1), _f32),
                   jax.ShapeDtypeStruct((1, 1), _f32)],
        scratch_shapes=[pltpu.VMEM((BS, H), _f32)],
    )(qe, qed, pid2, qa2, M2g, c2g2, qa_emb, qa_diff2, W_ihT, b_comb,
      W_hhT, bhn2)


_S_BLK = 8


def _student_body(m_col_ref, m_row_ref, q_col_ref, q_row_ref, qa_col_ref,
                  qa_row_ref, tgt_ref, pidsq_ref, pred_ref, loss_ref, cnt_ref):
    s = pl.program_id(0)
    m_col = m_col_ref[...]
    m_row = m_row_ref[...]
    q_col = q_col_ref[...]
    q_row = q_row_ref[...]
    qa_col = qa_col_ref[...]
    qa_row = qa_row_ref[...]

    mk_col = jnp.where(m_col >= 0.4, 1.0, m_col)
    mk_row = jnp.where(m_row >= 0.4, 1.0, m_row)
    i1c = mk_col == 1.0
    i0c = mk_col == 0.0
    i1r = mk_row == 1.0
    qa1c = qa_col == 1
    qa1r = qa_row == 1

    eq = q_col == q_row
    ri = lax.broadcasted_iota(_i32, (_S_BLK, L, L), 2)
    ci = lax.broadcasted_iota(_i32, (_S_BLK, L, L), 1)
    m_lt = eq & (ri < ci)
    eq_le = eq & (ri <= ci)

    w_mc = jnp.where(qa1c & i1c, 1.0, 0.0)
    w_mi = jnp.where((~qa1c) & i1c, 1.0 / 256.0, 0.0)
    w_nmc = jnp.where((~qa1c) & i0c, 1.0 / 65536.0, 0.0)
    w_packed = (w_mc + w_mi + w_nmc).reshape(_S_BLK, 1, L)
    v = jnp.sum(jnp.where(m_lt, w_packed, 0.0), axis=2, keepdims=True)
    aa = jnp.sum(jnp.where(m_lt, 1.0, 0.0), axis=2, keepdims=True) + 1.0
    mc = jnp.floor(v)
    r1 = (v - mc) * 256.0
    mi = jnp.floor(r1)
    nmc = (r1 - mi) * 256.0

    g_val = jnp.where(i1c & qa1c, mc / aa,
                      jnp.where((~i1c) & (~qa1c), 1.0 - nmc / aa, nmc / aa))
    s_val = mi / aa

    kf = lax.broadcasted_iota(
        _i32, (_S_BLK, 1, L), 2).astype(_f32) * 2.0
    g_enc = kf + g_val.reshape(_S_BLK, 1, L)
    s_enc = kf + s_val.reshape(_S_BLK, 1, L)
    set_g_r = (i1r & qa1r) | (~i1r)
    set_s_r = i1r & (~qa1r)
    ge = jnp.max(jnp.where(eq_le & set_g_r, g_enc, -1.0), axis=2,
                 keepdims=True)
    se = jnp.max(jnp.where(eq_le & set_s_r, s_enc, -1.0), axis=2,
                 keepdims=True)
    guess = jnp.where(ge < 0.0, 0.0, ge - 2.0 * jnp.floor(ge * 0.5))
    slip = jnp.where(se < 0.0, 0.0, se - 2.0 * jnp.floor(se * 0.5))

    res = (1.0 - slip) * (mk_col * guess + (1.0 - slip) * (1.0 - mk_col))
    pred_ref[...] = jax.nn.sigmoid(res)

    tgt = tgt_ref[...]
    maskl = tgt > -0.9
    sq = (res - tgt) * (res - tgt)

    @pl.when(s == 0)
    def _():
        loss_ref[...] = L2_CONST * pidsq_ref[...]
        cnt_ref[...] = jnp.zeros_like(cnt_ref)

    loss_ref[...] += jnp.sum(jnp.where(maskl, sq, 0.0)).reshape(1, 1)
    cnt_ref[...] += jnp.sum(maskl.astype(_i32)).reshape(1, 1)


def _student_call(m_col3, m_row3, q_col3, q_row3, qa_col3, qa_row3, tgt_col3,
                  pidsq):
    col = pl.BlockSpec((_S_BLK, L, 1), lambda s: (s, 0, 0))
    row = pl.BlockSpec((_S_BLK, 1, L), lambda s: (s, 0, 0))
    return pl.pallas_call(
        _student_body,
        grid=(BS // _S_BLK,),
        in_specs=[col, row, col, row, col, row, col,
                  pl.BlockSpec((1, 1), lambda s: (0, 0))],
        out_specs=[col,
                   pl.BlockSpec((1, 1), lambda s: (0, 0)),
                   pl.BlockSpec((1, 1), lambda s: (0, 0))],
        out_shape=[jax.ShapeDtypeStruct((BS, L, 1), _f32),
                   jax.ShapeDtypeStruct((1, 1), _f32),
                   jax.ShapeDtypeStruct((1, 1), _i32)],
    )(m_col3, m_row3, q_col3, q_row3, qa_col3, qa_row3, tgt_col3, pidsq)


def kernel(q_data, qa_data, matrix, target, pid_data, q_emb, qa_emb, diff_parm,
           q_emb_diff, qa_emb_diff, W_ih, W_hh, b_ih, b_hh, fc_W, fc_b):
    nq = q_emb.shape[0] - 1
    q_i = q_data.astype(_i32)
    qa = (qa_data.astype(_i32) - q_i) // nq
    qT = q_i.T.reshape(-1)
    pidT = pid_data.astype(_i32).T.reshape(-1)
    qm1 = qT - 1
    diff_flat = diff_parm.reshape(-1)

    M2, c2 = _m2_call(matrix, fc_W, fc_b.reshape(-1, 1))
    pid_e, qe, qed, M2g, c2g = _sc_gather(
        pidT, qT, qm1, diff_flat, q_emb, q_emb_diff, M2, c2.reshape(-1))

    b_comb = (b_ih + jnp.concatenate(
        [b_hh[:2 * H], jnp.zeros((H,), _f32)])).reshape(1, -1)
    m3, pidsq = _gru_call(qe, qed, pid_e.reshape(-1, 1), qa.T.reshape(-1, 1),
                          M2g, c2g.reshape(-1, 1), qa_emb, qa_emb_diff[:2],
                          W_ih.T, b_comb, W_hh.T, b_hh[2 * H:].reshape(1, H))

    m_bl = m3.reshape(L, BS).T
    preds3, loss, cnt = _student_call(
        m_bl.reshape(BS, L, 1), m_bl.reshape(BS, 1, L),
        q_i.reshape(BS, L, 1), q_i.reshape(BS, 1, L),
        qa.reshape(BS, L, 1), qa.reshape(BS, 1, L),
        target.reshape(BS, L, 1), pidsq)

    preds = preds3.reshape(-1)
    return loss[0, 0], preds, cnt[0, 0]


# speedup vs baseline: 1.1135x; 1.0035x over previous
import functools

import jax
import jax.numpy as jnp
from jax import lax
from jax.experimental import pallas as pl
from jax.experimental.pallas import tpu as pltpu
from jax.experimental.pallas import tpu_sc as plsc

BS, L, D, H = 64, 200, 128, 128
NTOK = BS * L
L2_CONST = 1e-05
NC, NS = 2, 16
NW = NC * NS
BPW = NTOK // NW

_f32 = jnp.float32
_i32 = jnp.int32


@functools.lru_cache(maxsize=1)
def _sc_gather_fn():
    mesh = plsc.VectorSubcoreMesh(core_axis_name="c", subcore_axis_name="s")

    @functools.partial(
        pl.kernel,
        mesh=mesh,
        out_type=[
            jax.ShapeDtypeStruct((NTOK,), _f32),
            jax.ShapeDtypeStruct((NTOK, D), _f32),
            jax.ShapeDtypeStruct((NTOK, D), _f32),
            jax.ShapeDtypeStruct((NTOK, D), _f32),
            jax.ShapeDtypeStruct((NTOK,), _f32),
        ],
        scratch_types=[
            pltpu.VMEM((BPW,), _i32),
            pltpu.VMEM((BPW,), _i32),
            pltpu.VMEM((BPW,), _i32),
            pltpu.VMEM((BPW,), _f32),
            pltpu.VMEM((BPW,), _f32),
            pltpu.VMEM((BPW, D), _f32),
            pltpu.VMEM((BPW, D), _f32),
            pltpu.SemaphoreType.DMA,
            pltpu.SemaphoreType.DMA,
            pltpu.SemaphoreType.DMA,
            pltpu.SemaphoreType.DMA,
            pltpu.SemaphoreType.DMA,
            pltpu.SemaphoreType.DMA,
            pltpu.SemaphoreType.DMA,
            pltpu.SemaphoreType.DMA,
        ],
    )
    def _gather(pid_hbm, q_hbm, qm1_hbm, diff_hbm, qemb_hbm, qed_hbm, m2_hbm,
                c2_hbm, pid_out, qe_out, qed_out, m2g_out, c2g_out,
                idx_p, idx_q, idx_m, vals1, vals2, rows1, rows2,
                s_ip, s_iq, s_im, s_a, s_b, s_c, s_d, s_e):
        wid = lax.axis_index("s") * NC + lax.axis_index("c")
        base = wid * BPW
        sl = pl.ds(base, BPW)
        d_ip = pltpu.async_copy(pid_hbm.at[sl], idx_p, s_ip)
        d_iq = pltpu.async_copy(q_hbm.at[sl], idx_q, s_iq)
        d_im = pltpu.async_copy(qm1_hbm.at[sl], idx_m, s_im)
        d_ip.wait()
        g_a = pltpu.async_copy(diff_hbm.at[idx_p], vals1, s_a)
        d_iq.wait()
        g_b = pltpu.async_copy(qemb_hbm.at[idx_q], rows1, s_b)
        g_c = pltpu.async_copy(qed_hbm.at[idx_q], rows2, s_c)
        d_im.wait()
        g_e = pltpu.async_copy(c2_hbm.at[idx_m], vals2, s_e)
        g_a.wait()
        pltpu.sync_copy(vals1, pid_out.at[sl])
        g_b.wait()
        pltpu.sync_copy(rows1, qe_out.at[sl])
        g_d = pltpu.async_copy(m2_hbm.at[idx_m], rows1, s_d)
        g_c.wait()
        pltpu.sync_copy(rows2, qed_out.at[sl])
        g_e.wait()
        pltpu.sync_copy(vals2, c2g_out.at[sl])
        g_d.wait()
        pltpu.sync_copy(rows1, m2g_out.at[sl])

    return _gather


def _sc_gather(pidT, qT, qm1, diff_flat, q_emb, q_emb_diff, M2, c2_flat):
    return _sc_gather_fn()(pidT, qT, qm1, diff_flat, q_emb, q_emb_diff,
                           M2, c2_flat)


def _m2_body(mat_ref, fcw_ref, fcb_ref, m2_ref, c2_ref):
    mat = mat_ref[...]
    m2_ref[...] = jnp.dot(mat, fcw_ref[...], preferred_element_type=_f32)
    c2_ref[...] = jnp.dot(mat, fcb_ref[...], preferred_element_type=_f32)


def _m2_call(matrix, fc_W, fc_b2):
    nq = matrix.shape[0]
    return pl.pallas_call(
        _m2_body,
        out_shape=[jax.ShapeDtypeStruct((nq, H), _f32),
                   jax.ShapeDtypeStruct((nq, 1), _f32)],
    )(matrix, fc_W, fc_b2)


_T_BLK = 25
_RB = _T_BLK * BS


def _gru_body(qe_ref, qed_ref, pid_ref, qa_ref, m2g_ref, c2g_ref, qaemb_ref,
              qadiff_ref, wih_ref, bcomb_ref, whh_ref, bhn_ref,
              m_ref, pidsq_ref, h_ref):
    @pl.when(pl.program_id(0) == 0)
    def _():
        h_ref[...] = jnp.zeros_like(h_ref)
        pidsq_ref[...] = jnp.zeros_like(pidsq_ref)

    qe = qe_ref[...]
    qed = qed_ref[...]
    pid = pid_ref[...]
    qa1 = qa_ref[...] == 1
    row0 = qaemb_ref[0:1, :]
    row1 = qaemb_ref[1:2, :]
    d0 = qadiff_ref[0:1, :]
    d1 = qadiff_ref[1:2, :]
    qa_row = jnp.where(qa1, row1, row0)
    qa_diff = jnp.where(qa1, d1, d0)
    q_full = qe + pid * qed
    qa_full = qe + qa_row + pid * qa_diff
    x = jnp.concatenate([qa_full, q_full], axis=1)
    gi_all = jnp.dot(x, wih_ref[...],
                     preferred_element_type=_f32) + bcomb_ref[...]
    pidsq_ref[...] += jnp.sum(pid * pid).reshape(1, 1)

    whh = whh_ref[...]
    bhn = bhn_ref[...]
    hb = BS // 2
    h1 = h_ref[0:hb, :]
    h2 = h_ref[hb:BS, :]

    def _step(h, gh, gi):
        r = jax.nn.sigmoid(gi[:, :H] + gh[:, :H])
        z = jax.nn.sigmoid(gi[:, H:2 * H] + gh[:, H:2 * H])
        n = jnp.tanh(gi[:, 2 * H:] + r * (gh[:, 2 * H:] + bhn))
        return (1.0 - z) * n + z * h

    for j in range(_T_BLK):
        base = j * BS
        gh1 = jnp.dot(h1, whh, preferred_element_type=_f32)
        gh2 = jnp.dot(h2, whh, preferred_element_type=_f32)
        h1 = _step(h1, gh1, gi_all[base:base + hb])
        h2 = _step(h2, gh2, gi_all[base + hb:base + BS])
        m_ref[base:base + hb] = (
            jnp.sum(h1 * m2g_ref[base:base + hb], axis=1, keepdims=True)
            + c2g_ref[base:base + hb])
        m_ref[base + hb:base + BS] = (
            jnp.sum(h2 * m2g_ref[base + hb:base + BS], axis=1, keepdims=True)
            + c2g_ref[base + hb:base + BS])
    h_ref[0:hb, :] = h1
    h_ref[hb:BS, :] = h2


def _gru_call(qe, qed, pid2, qa2, M2g, c2g2, qa_emb, qa_diff2, W_ihT, b_comb,
              W_hhT, bhn2):
    blk = lambda w: pl.BlockSpec((_RB, w), lambda t: (t, 0))
    full = lambda a, b: pl.BlockSpec((a, b), lambda t: (0, 0))
    return pl.pallas_call(
        _gru_body,
        grid=(L // _T_BLK,),
        in_specs=[
            blk(D), blk(D), blk(1), blk(1), blk(D), blk(1),
            full(2, D), full(2, D), full(2 * D, 3 * H), full(1, 3 * H),
            full(H, 3 * H), full(1, H),
        ],
        out_specs=[
            blk(1),
            pl.BlockSpec((1, 1), lambda t: (0, 0)),
        ],
        out_shape=[jax.ShapeDtypeStruct((NTOK, 1), _f32),
                   jax.ShapeDtypeStruct((1, 1), _f32)],
        scratch_shapes=[pltpu.VMEM((BS, H), _f32)],
    )(qe, qed, pid2, qa2, M2g, c2g2, qa_emb, qa_diff2, W_ihT, b_comb,
      W_hhT, bhn2)


_S_BLK = 8


def _student_body(m_col_ref, m_row_ref, q_col_ref, q_row_ref, qa_col_ref,
                  qa_row_ref, tgt_ref, pidsq_ref, pred_ref, loss_ref, cnt_ref):
    s = pl.program_id(0)
    m_col = m_col_ref[...]
    m_row = m_row_ref[...]
    q_col = q_col_ref[...]
    q_row = q_row_ref[...]
    qa_col = qa_col_ref[...]
    qa_row = qa_row_ref[...]

    mk_col = jnp.where(m_col >= 0.4, 1.0, m_col)
    mk_row = jnp.where(m_row >= 0.4, 1.0, m_row)
    i1c = mk_col == 1.0
    i0c = mk_col == 0.0
    i1r = mk_row == 1.0
    qa1c = qa_col == 1
    qa1r = qa_row == 1

    eq = q_col == q_row
    ri = lax.broadcasted_iota(_i32, (_S_BLK, L, L), 2)
    ci = lax.broadcasted_iota(_i32, (_S_BLK, L, L), 1)
    m_lt = eq & (ri < ci)
    eq_le = eq & (ri <= ci)

    w_mc = jnp.where(qa1c & i1c, 1.0, 0.0)
    w_mi = jnp.where((~qa1c) & i1c, 1.0 / 256.0, 0.0)
    w_nmc = jnp.where((~qa1c) & i0c, 1.0 / 65536.0, 0.0)
    w_packed = (w_mc + w_mi + w_nmc).reshape(_S_BLK, 1, L)
    v = jnp.sum(jnp.where(m_lt, w_packed, 0.0), axis=2, keepdims=True)
    aa = jnp.sum(jnp.where(m_lt, 1.0, 0.0), axis=2, keepdims=True) + 1.0
    mc = jnp.floor(v)
    r1 = (v - mc) * 256.0
    mi = jnp.floor(r1)
    nmc = (r1 - mi) * 256.0

    g_val = jnp.where(i1c & qa1c, mc / aa,
                      jnp.where((~i1c) & (~qa1c), 1.0 - nmc / aa, nmc / aa))
    s_val = mi / aa

    kf = lax.broadcasted_iota(
        _i32, (_S_BLK, 1, L), 2).astype(_f32) * 2.0
    g_enc = kf + g_val.reshape(_S_BLK, 1, L)
    s_enc = kf + s_val.reshape(_S_BLK, 1, L)
    set_g_r = (i1r & qa1r) | (~i1r)
    set_s_r = i1r & (~qa1r)
    ge = jnp.max(jnp.where(eq_le & set_g_r, g_enc, -1.0), axis=2,
                 keepdims=True)
    se = jnp.max(jnp.where(eq_le & set_s_r, s_enc, -1.0), axis=2,
                 keepdims=True)
    guess = jnp.where(ge < 0.0, 0.0, ge - 2.0 * jnp.floor(ge * 0.5))
    slip = jnp.where(se < 0.0, 0.0, se - 2.0 * jnp.floor(se * 0.5))

    res = (1.0 - slip) * (mk_col * guess + (1.0 - slip) * (1.0 - mk_col))
    pred_ref[...] = jax.nn.sigmoid(res)

    tgt = tgt_ref[...]
    maskl = tgt > -0.9
    sq = (res - tgt) * (res - tgt)

    @pl.when(s == 0)
    def _():
        loss_ref[...] = L2_CONST * pidsq_ref[...]
        cnt_ref[...] = jnp.zeros_like(cnt_ref)

    loss_ref[...] += jnp.sum(jnp.where(maskl, sq, 0.0)).reshape(1, 1)
    cnt_ref[...] += jnp.sum(maskl.astype(_i32)).reshape(1, 1)


def _student_call(m_col3, m_row3, q_col3, q_row3, qa_col3, qa_row3, tgt_col3,
                  pidsq):
    col = pl.BlockSpec((_S_BLK, L, 1), lambda s: (s, 0, 0))
    row = pl.BlockSpec((_S_BLK, 1, L), lambda s: (s, 0, 0))
    return pl.pallas_call(
        _student_body,
        grid=(BS // _S_BLK,),
        in_specs=[col, row, col, row, col, row, col,
                  pl.BlockSpec((1, 1), lambda s: (0, 0))],
        out_specs=[col,
                   pl.BlockSpec((1, 1), lambda s: (0, 0)),
                   pl.BlockSpec((1, 1), lambda s: (0, 0))],
        out_shape=[jax.ShapeDtypeStruct((BS, L, 1), _f32),
                   jax.ShapeDtypeStruct((1, 1), _f32),
                   jax.ShapeDtypeStruct((1, 1), _i32)],
    )(m_col3, m_row3, q_col3, q_row3, qa_col3, qa_row3, tgt_col3, pidsq)


def kernel(q_data, qa_data, matrix, target, pid_data, q_emb, qa_emb, diff_parm,
           q_emb_diff, qa_emb_diff, W_ih, W_hh, b_ih, b_hh, fc_W, fc_b):
    nq = q_emb.shape[0] - 1
    q_i = q_data.astype(_i32)
    qa = (qa_data.astype(_i32) - q_i) // nq
    qT = q_i.T.reshape(-1)
    pidT = pid_data.astype(_i32).T.reshape(-1)
    qm1 = qT - 1
    diff_flat = diff_parm.reshape(-1)

    M2, c2 = _m2_call(matrix, fc_W, fc_b.reshape(-1, 1))
    pid_e, qe, qed, M2g, c2g = _sc_gather(
        pidT, qT, qm1, diff_flat, q_emb, q_emb_diff, M2, c2.reshape(-1))

    b_comb = (b_ih + jnp.concatenate(
        [b_hh[:2 * H], jnp.zeros((H,), _f32)])).reshape(1, -1)
    m3, pidsq = _gru_call(qe, qed, pid_e.reshape(-1, 1), qa.T.reshape(-1, 1),
                          M2g, c2g.reshape(-1, 1), qa_emb, qa_emb_diff[:2],
                          W_ih.T, b_comb, W_hh.T, b_hh[2 * H:].reshape(1, H))

    m_bl = m3.reshape(L, BS).T
    preds3, loss, cnt = _student_call(
        m_bl.reshape(BS, L, 1), m_bl.reshape(BS, 1, L),
        q_i.reshape(BS, L, 1), q_i.reshape(BS, 1, L),
        qa.reshape(BS, L, 1), qa.reshape(BS, 1, L),
        target.reshape(BS, L, 1), pidsq)

    preds = preds3.reshape(-1)
    return loss[0, 0], preds, cnt[0, 0]
